# baseline (speedup 1.0000x reference)
import functools

import jax
import jax.numpy as jnp
from jax import lax
from jax.experimental import pallas as pl
from jax.experimental.pallas import tpu as pltpu
from jax.experimental.pallas import tpu_sc as plsc

CHUNK_ROWS = 8
NBUF = 4
PAD_F = 128
GATHER_F = 32


@functools.lru_cache(maxsize=None)
def _build_gather(batch: int, fields: int, dim: int, cr: int, nbuf: int):
    mesh = plsc.VectorSubcoreMesh(core_axis_name="c", subcore_axis_name="s")
    n_workers = mesh.num_cores * mesh.num_subcores
    assert batch % n_workers == 0
    rows_per_w = batch // n_workers
    assert rows_per_w % cr == 0
    n_chunks = rows_per_w // cr
    assert n_chunks % nbuf == 0
    assert fields <= GATHER_F <= PAD_F

    @functools.partial(
        pl.kernel,
        out_type=jax.ShapeDtypeStruct((batch, fields, dim), jnp.float32),
        mesh=mesh,
        scratch_types=[
            [pltpu.VMEM((cr, PAD_F), jnp.int32) for _ in range(nbuf)],
            [pltpu.VMEM((cr, GATHER_F, dim), jnp.float32)
             for _ in range(nbuf)],
            [pltpu.SemaphoreType.DMA for _ in range(nbuf)],
            [pltpu.SemaphoreType.DMA for _ in range(nbuf)],
        ],
        compiler_params=pltpu.CompilerParams(use_tc_tiling_on_sc=False),
    )
    def gather_kernel(idx_hbm, table_hbm, out_hbm,
                      blk_v, rows_v, gsem, ssem):
        wid = lax.axis_index("s") * mesh.num_cores + lax.axis_index("c")
        base = wid * rows_per_w

        def stage(b, g):
            pltpu.sync_copy(idx_hbm.at[pl.ds(base + g * cr, cr), :],
                            blk_v[b])
            for r in range(cr):
                pltpu.async_copy(
                    table_hbm.at[blk_v[b].at[r, pl.ds(0, GATHER_F)]],
                    rows_v[b].at[r],
                    gsem[b])

        def drain_gather(b):
            for r in range(cr):
                pltpu.make_async_copy(
                    table_hbm.at[blk_v[b].at[r, pl.ds(0, GATHER_F)]],
                    rows_v[b].at[r],
                    gsem[b]).wait()

        def start_store(b, g):
            for r in range(cr):
                pltpu.async_copy(rows_v[b].at[r, pl.ds(0, fields), :],
                                 out_hbm.at[base + g * cr + r],
                                 ssem[b])

        def drain_store(b, g):
            for r in range(cr):
                pltpu.make_async_copy(rows_v[b].at[r, pl.ds(0, fields), :],
                                      out_hbm.at[base + g * cr + r],
                                      ssem[b]).wait()

        for b in range(nbuf):
            stage(b, b)

        def body(grp, carry):
            g0 = grp * nbuf
            for b in range(nbuf):
                g = g0 + b
                drain_gather(b)
                start_store(b, g)

                @pl.when(g + nbuf < n_chunks)
                def _():
                    drain_store(b, g)
                    stage(b, g + nbuf)

            return carry

        lax.fori_loop(0, n_chunks // nbuf, body, 0)

        for b in range(nbuf):
            drain_store(b, n_chunks - nbuf + b)

    return gather_kernel


def kernel(input, weight):
    b, f = input.shape
    idx_pad = jnp.pad(input.astype(jnp.int32), ((0, 0), (0, PAD_F - f)))
    return _build_gather(b, f, weight.shape[1], CHUNK_ROWS, NBUF)(
        idx_pad, weight)


# speedup vs baseline: 3.0547x; 3.0547x over previous
import functools

import jax
import jax.numpy as jnp
from jax import lax
from jax.experimental import pallas as pl
from jax.experimental.pallas import tpu as pltpu
from jax.experimental.pallas import tpu_sc as plsc

CHUNK_ROWS = 16
NBUF = 4
PAD_F = 128
LANES = 16


@functools.lru_cache(maxsize=None)
def _build_gather(batch: int, fields: int, dim: int, cr: int, nbuf: int):
    mesh = plsc.VectorSubcoreMesh(core_axis_name="c", subcore_axis_name="s")
    n_workers = mesh.num_cores * mesh.num_subcores
    assert batch % n_workers == 0
    rows_per_w = batch // n_workers
    assert rows_per_w % cr == 0
    n_chunks = rows_per_w // cr
    assert n_chunks % nbuf == 0
    chunk = cr * fields
    assert chunk % LANES == 0 and chunk % 8 == 0
    n_vecs = chunk // LANES

    @functools.partial(
        pl.kernel,
        out_type=jax.ShapeDtypeStruct((batch * fields, dim), jnp.float32),
        mesh=mesh,
        scratch_types=[
            [pltpu.VMEM((cr, PAD_F), jnp.int32) for _ in range(nbuf)],
            [pltpu.VMEM((chunk,), jnp.int32) for _ in range(nbuf)],
            [pltpu.VMEM((chunk, dim), jnp.float32) for _ in range(nbuf)],
            [pltpu.SemaphoreType.DMA for _ in range(nbuf)],
            [pltpu.SemaphoreType.DMA for _ in range(nbuf)],
        ],
        compiler_params=pltpu.CompilerParams(use_tc_tiling_on_sc=False,
                                             needs_layout_passes=False),
    )
    def gather_kernel(idx_hbm, table_hbm, out_hbm,
                      blk_v, idx_v, rows_v, gsem, ssem):
        wid = lax.axis_index("s") * mesh.num_cores + lax.axis_index("c")
        base = wid * rows_per_w

        def stage(b, g):
            pltpu.sync_copy(idx_hbm.at[pl.ds(base + g * cr, cr), :],
                            blk_v[b])
            magic = -(-(1 << 16) // fields)
            lane = lax.iota(jnp.int32, LANES)
            for v in range(n_vecs):
                k = lane + v * LANES
                r = lax.shift_right_logical(k * magic, 16)
                c = k - r * fields
                idx_v[b][pl.ds(v * LANES, LANES)] = plsc.load_gather(
                    blk_v[b], [r, c])
            pltpu.async_copy(table_hbm.at[idx_v[b]], rows_v[b], gsem[b])

        def out_slice(g):
            return out_hbm.at[pl.ds((base + g * cr) * fields, chunk)]

        for b in range(nbuf):
            stage(b, b)

        def body(grp, carry):
            g0 = grp * nbuf
            for b in range(nbuf):
                g = g0 + b
                pltpu.make_async_copy(table_hbm.at[idx_v[b]], rows_v[b],
                                      gsem[b]).wait()
                pltpu.async_copy(rows_v[b], out_slice(g), ssem[b])

                @pl.when(g + nbuf < n_chunks)
                def _():
                    pltpu.make_async_copy(rows_v[b], out_slice(g),
                                          ssem[b]).wait()
                    stage(b, g + nbuf)

            return carry

        lax.fori_loop(0, n_chunks // nbuf, body, 0)

        for b in range(nbuf):
            g = n_chunks - nbuf + b
            pltpu.make_async_copy(rows_v[b], out_slice(g), ssem[b]).wait()

    return gather_kernel


def kernel(input, weight):
    b, f = input.shape
    d = weight.shape[1]
    idx_pad = jnp.pad(input.astype(jnp.int32), ((0, 0), (0, PAD_F - f)))
    out = _build_gather(b, f, d, CHUNK_ROWS, NBUF)(idx_pad, weight)
    return out.reshape(b, f, d)


# speedup vs baseline: 3.0904x; 1.0117x over previous
import functools

import jax
import jax.numpy as jnp
from jax import lax
from jax.experimental import pallas as pl
from jax.experimental.pallas import tpu as pltpu
from jax.experimental.pallas import tpu_sc as plsc

NBUF = 2
LANES = 16


@functools.lru_cache(maxsize=None)
def _build_gather(batch: int, fields: int, dim: int, nbuf: int):
    mesh = plsc.VectorSubcoreMesh(core_axis_name="c", subcore_axis_name="s")
    n_workers = mesh.num_cores * mesh.num_subcores
    assert batch % n_workers == 0
    rows = batch // n_workers
    assert rows % LANES == 0
    n_vecs = rows // LANES

    @functools.partial(
        pl.kernel,
        out_type=jax.ShapeDtypeStruct((batch, fields, dim), jnp.float32),
        mesh=mesh,
        scratch_types=[
            pltpu.VMEM((rows, fields), jnp.float32),
            [pltpu.VMEM((rows,), jnp.int32) for _ in range(nbuf)],
            [pltpu.VMEM((rows, dim), jnp.float32) for _ in range(nbuf)],
            [pltpu.SemaphoreType.DMA for _ in range(nbuf)],
            [pltpu.SemaphoreType.DMA for _ in range(nbuf)],
        ],
        compiler_params=pltpu.CompilerParams(use_tc_tiling_on_sc=False,
                                             needs_layout_passes=False),
    )
    def gather_kernel(idx_hbm, table_hbm, out_hbm,
                      blk_v, idx_v, rows_v, gsem, ssem):
        wid = lax.axis_index("s") * mesh.num_cores + lax.axis_index("c")
        base = wid * rows

        pltpu.sync_copy(idx_hbm.at[pl.ds(base, rows), :], blk_v)

        lane = lax.iota(jnp.int32, LANES)

        def stage(b, f):
            c = jnp.full((LANES,), 0, jnp.int32) + f
            for v in range(n_vecs):
                r = lane + v * LANES
                idx_v[b][pl.ds(v * LANES, LANES)] = plsc.bitcast(
                    plsc.load_gather(blk_v, [r, c]), jnp.int32)
            pltpu.async_copy(table_hbm.at[idx_v[b]], rows_v[b], gsem[b])

        def out_slice(f):
            return out_hbm.at[pl.ds(base, rows), f, :]

        for b in range(nbuf):
            stage(b, b)

        def body(grp, carry):
            f0 = grp * nbuf
            for b in range(nbuf):
                f = f0 + b
                pltpu.make_async_copy(table_hbm.at[idx_v[b]], rows_v[b],
                                      gsem[b]).wait()
                pltpu.async_copy(rows_v[b], out_slice(f), ssem[b])

                @pl.when(f + nbuf < fields)
                def _():
                    pltpu.make_async_copy(rows_v[b], out_slice(f),
                                          ssem[b]).wait()
                    stage(b, f + nbuf)

            return carry

        lax.fori_loop(0, fields // nbuf, body, 0)

        rem = fields % nbuf
        for b in range(rem):
            f = (fields // nbuf) * nbuf + b
            pltpu.make_async_copy(table_hbm.at[idx_v[b]], rows_v[b],
                                  gsem[b]).wait()
            pltpu.async_copy(rows_v[b], out_slice(f), ssem[b])
        for b in range(nbuf):
            f = fields - nbuf + b
            pltpu.make_async_copy(rows_v[b], out_slice(f), ssem[b]).wait()

    return gather_kernel


def kernel(input, weight):
    b, f = input.shape
    d = weight.shape[1]
    idx_f = jax.lax.bitcast_convert_type(input.astype(jnp.int32),
                                         jnp.float32)
    return _build_gather(b, f, d, NBUF)(idx_f, weight)


# speedup vs baseline: 3.1022x; 1.0038x over previous
import functools

import jax
import jax.numpy as jnp
from jax import lax
from jax.experimental import pallas as pl
from jax.experimental.pallas import tpu as pltpu
from jax.experimental.pallas import tpu_sc as plsc

NBUF = 2


@functools.lru_cache(maxsize=None)
def _build_gather(batch: int, fields: int, dim: int, nbuf: int):
    mesh = plsc.VectorSubcoreMesh(core_axis_name="c", subcore_axis_name="s")
    n_workers = mesh.num_cores * mesh.num_subcores
    assert batch % n_workers == 0
    rows = batch // n_workers

    @functools.partial(
        pl.kernel,
        out_type=jax.ShapeDtypeStruct((batch, fields, dim), jnp.float32),
        mesh=mesh,
        scratch_types=[
            [pltpu.VMEM((rows,), jnp.int32) for _ in range(nbuf)],
            [pltpu.VMEM((rows, dim), jnp.float32) for _ in range(nbuf)],
            [pltpu.SemaphoreType.DMA for _ in range(nbuf)],
            [pltpu.SemaphoreType.DMA for _ in range(nbuf)],
            [pltpu.SemaphoreType.DMA for _ in range(nbuf)],
        ],
        compiler_params=pltpu.CompilerParams(use_tc_tiling_on_sc=False),
    )
    def gather_kernel(idx_hbm, table_hbm, out_hbm,
                      idx_v, rows_v, isem, gsem, ssem):
        wid = lax.axis_index("s") * mesh.num_cores + lax.axis_index("c")
        base = wid * rows

        def stage(b, f):
            pltpu.async_copy(idx_hbm.at[f, pl.ds(base, rows)],
                             idx_v[b], isem[b]).wait()
            pltpu.async_copy(table_hbm.at[idx_v[b]], rows_v[b], gsem[b])

        def out_slice(f):
            return out_hbm.at[pl.ds(base, rows), f, :]

        for b in range(nbuf):
            stage(b, b)

        def body(grp, carry):
            f0 = grp * nbuf
            for b in range(nbuf):
                f = f0 + b
                pltpu.make_async_copy(table_hbm.at[idx_v[b]], rows_v[b],
                                      gsem[b]).wait()
                pltpu.async_copy(rows_v[b], out_slice(f), ssem[b])

                @pl.when(f + nbuf < fields)
                def _():
                    pltpu.make_async_copy(rows_v[b], out_slice(f),
                                          ssem[b]).wait()
                    stage(b, f + nbuf)

            return carry

        lax.fori_loop(0, fields // nbuf, body, 0)

        rem = fields % nbuf
        for b in range(rem):
            f = (fields // nbuf) * nbuf + b
            pltpu.make_async_copy(table_hbm.at[idx_v[b]], rows_v[b],
                                  gsem[b]).wait()
            pltpu.async_copy(rows_v[b], out_slice(f), ssem[b])
        for b in range(nbuf):
            f = fields - nbuf + b
            pltpu.make_async_copy(rows_v[b], out_slice(f), ssem[b]).wait()

    return gather_kernel


def kernel(input, weight):
    b, f = input.shape
    return _build_gather(b, f, weight.shape[1], NBUF)(
        input.astype(jnp.int32).T, weight)
